# Initial kernel scaffold; baseline (speedup 1.0000x reference)
import jax
import jax.numpy as jnp
from jax.experimental import pallas as pl


def kernel(indices, state):
    raise NotImplementedError("write your pallas kernel here")


# speedup vs baseline: 1.7310x; 1.7310x over previous
import functools

import jax
import jax.numpy as jnp
from jax import lax
from jax.experimental import pallas as pl
from jax.experimental.pallas import tpu as pltpu
from jax.experimental.pallas import tpu_sc as plsc

M, D = 16384, 2048
NC, NS = 2, 16
NW = NC * NS
ROWS_PER_W = M // NW
CHUNK = 16
NCHUNKS = ROWS_PER_W // CHUNK
NBUF = 2

_mesh = plsc.VectorSubcoreMesh(
    core_axis_name="c", subcore_axis_name="s", num_cores=NC, num_subcores=NS
)


@functools.partial(
    pl.kernel,
    out_type=jax.ShapeDtypeStruct((M, D), jnp.float32),
    mesh=_mesh,
    scratch_types=[
        pltpu.VMEM((ROWS_PER_W,), jnp.int32),
        pltpu.VMEM((NBUF, CHUNK, D), jnp.float32),
        pltpu.SemaphoreType.DMA,
        pltpu.SemaphoreType.DMA,
        pltpu.SemaphoreType.DMA,
        pltpu.SemaphoreType.DMA,
    ],
)
def _gather_rows(idx_hbm, table_hbm, out_hbm, idx_v, rows_v, g0, g1, w0, w1):
    gsems = (g0, g1)
    wsems = (w0, w1)
    wid = lax.axis_index("s") * NC + lax.axis_index("c")
    base = wid * ROWS_PER_W

    pltpu.sync_copy(idx_hbm.at[pl.ds(base, ROWS_PER_W)], idx_v)

    def start_gather(c, b):
        idx_vec = idx_v[pl.ds(c * CHUNK, CHUNK)]
        pltpu.async_copy(table_hbm.at[idx_vec], rows_v.at[b], gsems[b])

    def wait_gather(b):
        pltpu.make_async_copy(
            table_hbm.at[pl.ds(0, CHUNK)], rows_v.at[b], gsems[b]
        ).wait()

    def start_write(c, b):
        pltpu.async_copy(
            rows_v.at[b], out_hbm.at[pl.ds(base + c * CHUNK, CHUNK)], wsems[b]
        )

    def wait_write(b):
        pltpu.make_async_copy(
            rows_v.at[b], out_hbm.at[pl.ds(0, CHUNK)], wsems[b]
        ).wait()

    start_gather(0, 0)

    @pl.loop(0, NCHUNKS, step=NBUF)
    def _(c0):
        for i in range(NBUF):
            c = c0 + i
            b = i
            b2 = (i + 1) % NBUF
            wait_gather(b)

            @pl.when(c >= 1)
            def _():
                wait_write(b2)

            @pl.when(c + 1 < NCHUNKS)
            def _():
                start_gather(c + 1, b2)

            start_write(c, b)

    wait_write((NCHUNKS - 1) % NBUF)


def kernel(indices, state):
    return _gather_rows(indices, state)


# speedup vs baseline: 1.7800x; 1.0283x over previous
import functools

import jax
import jax.numpy as jnp
from jax import lax
from jax.experimental import pallas as pl
from jax.experimental.pallas import tpu as pltpu
from jax.experimental.pallas import tpu_sc as plsc

M, D = 16384, 2048
NC, NS = 2, 16
NW = NC * NS
ROWS_PER_W = M // NW
CHUNK = 16
NCHUNKS = ROWS_PER_W // CHUNK
NBUF = 3

_mesh = plsc.VectorSubcoreMesh(
    core_axis_name="c", subcore_axis_name="s", num_cores=NC, num_subcores=NS
)


@functools.partial(
    pl.kernel,
    out_type=jax.ShapeDtypeStruct((M, D), jnp.float32),
    mesh=_mesh,
    scratch_types=[
        pltpu.VMEM((ROWS_PER_W,), jnp.int32),
        pltpu.VMEM((NBUF, CHUNK, D), jnp.float32),
        pltpu.SemaphoreType.DMA,
        pltpu.SemaphoreType.DMA,
        pltpu.SemaphoreType.DMA,
        pltpu.SemaphoreType.DMA,
        pltpu.SemaphoreType.DMA,
        pltpu.SemaphoreType.DMA,
    ],
)
def _gather_rows(idx_hbm, table_hbm, out_hbm, idx_v, rows_v,
                 g0, g1, g2, w0, w1, w2):
    gsems = (g0, g1, g2)
    wsems = (w0, w1, w2)
    wid = lax.axis_index("s") * NC + lax.axis_index("c")
    base = wid * ROWS_PER_W

    pltpu.sync_copy(idx_hbm.at[pl.ds(base, ROWS_PER_W)], idx_v)

    def start_gather(c, b):
        idx_vec = idx_v[pl.ds(c * CHUNK, CHUNK)]
        pltpu.async_copy(table_hbm.at[idx_vec], rows_v.at[b], gsems[b])

    def wait_gather(b):
        pltpu.make_async_copy(
            table_hbm.at[pl.ds(0, CHUNK)], rows_v.at[b], gsems[b]
        ).wait()

    def start_write(c, b):
        pltpu.async_copy(
            rows_v.at[b], out_hbm.at[pl.ds(base + c * CHUNK, CHUNK)], wsems[b]
        )

    def wait_write(b):
        pltpu.make_async_copy(
            rows_v.at[b], out_hbm.at[pl.ds(0, CHUNK)], wsems[b]
        ).wait()

    def step(c, b):
        b2 = (b + 2) % NBUF
        wait_gather(b)

        @pl.when(c >= 1)
        def _():
            wait_write(b2)

        @pl.when(c + 2 < NCHUNKS)
        def _():
            start_gather(c + 2, b2)

        start_write(c, b)

    start_gather(0, 0)
    start_gather(1, 1)

    _MAIN = (NCHUNKS // NBUF) * NBUF

    @pl.loop(0, _MAIN, step=NBUF)
    def _(c0):
        for i in range(NBUF):
            step(c0 + i, i)

    for c in range(_MAIN, NCHUNKS):
        step(c, c % NBUF)

    wait_write((NCHUNKS - 1) % NBUF)


def kernel(indices, state):
    return _gather_rows(indices, state)
